# baseline (speedup 1.0000x reference)
import functools

import jax
import jax.numpy as jnp
from jax.experimental import pallas as pl
from jax.experimental.pallas import tpu as pltpu

_N, _D, _H, _L = 4096, 512, 128, 15
_NC, _ADJ, _K = 15, 32, 3
_LP = 16


def _hilo(u, w):
    hi = u.astype(jnp.bfloat16)
    lo = (u - hi.astype(jnp.float32)).astype(jnp.bfloat16)
    return jnp.concatenate([hi, lo], axis=1)


_NB = 32
_BM = _N // _NB


def _enc_kernel(a_ref, x_ref, w1c_ref, b1c_ref, w2c_ref, b2c_ref, ones_ref,
                z_ref,
                b_vm, g_vm, dinv_vm, ua_vm, ub_vm, g2_vm):
    s = pl.program_id(0)

    @pl.when(s < _NB)
    def _scan():
        a = a_ref[...]
        maskb = (a != 0.0).astype(jnp.bfloat16)
        rows = pl.ds(jnp.minimum(s, _NB - 1) * _BM, _BM)
        b_vm[rows, :] = maskb
        deg = jnp.dot(maskb, ones_ref[...], preferred_element_type=jnp.float32)
        dinvb = jnp.where(deg > 0.0, jax.lax.rsqrt(deg), 0.0)
        dinv_vm[rows, :] = dinvb
        g = (jnp.dot(x_ref[...], w1c_ref[...], preferred_element_type=jnp.float32)
             + b1c_ref[...][None, :])
        g_vm[rows, :] = g[:, :3 * _H]
        ua_vm[rows, :] = _hilo(dinvb * g[:, 3 * _H:], _H)

    @pl.when(s == _NB)
    def _hops():
        cm = 1024
        nch = _N // cm
        for k, src, dst in ((2, ua_vm, ub_vm), (1, ub_vm, ua_vm)):
            for c in range(nch):
                rows = pl.ds(c * cm, cm)
                sf = jnp.dot(b_vm[rows, :], src[...],
                             preferred_element_type=jnp.float32)
                sw = sf[:, :_H] + sf[:, _H:]
                dinv = dinv_vm[rows, :]
                p = g_vm[rows, :][:, k * _H:(k + 1) * _H] + dinv * sw
                dst[rows, :] = _hilo(dinv * p, _H)
        for c in range(nch):
            rows = pl.ds(c * cm, cm)
            sf = jnp.dot(b_vm[rows, :], ua_vm[...],
                         preferred_element_type=jnp.float32)
            sw = sf[:, :_H] + sf[:, _H:]
            dinv = dinv_vm[rows, :]
            h = jnp.maximum(g_vm[rows, :][:, :_H] + dinv * sw, 0.0)
            g2 = (jnp.dot(h, w2c_ref[...], preferred_element_type=jnp.float32)
                  + b2c_ref[...][None, :])
            g2_vm[rows, :] = g2
            ub_vm[rows, pl.ds(0, 2 * _LP)] = _hilo(
                dinv[:, :_LP] * g2[:, 3 * _LP:], _LP)
        for k, so, do in ((2, 0, 2 * _LP), (1, 2 * _LP, 0)):
            for c in range(nch):
                rows = pl.ds(c * cm, cm)
                sf = jnp.dot(b_vm[rows, :], ub_vm[:, pl.ds(so, 2 * _LP)],
                             preferred_element_type=jnp.float32)
                sw = sf[:, :_LP] + sf[:, _LP:]
                dinv = dinv_vm[rows, :][:, :_LP]
                p = g2_vm[rows, :][:, k * _LP:(k + 1) * _LP] + dinv * sw
                ub_vm[rows, pl.ds(do, 2 * _LP)] = _hilo(dinv * p, _LP)
        for c in range(nch):
            rows = pl.ds(c * cm, cm)
            sf = jnp.dot(b_vm[rows, :], ub_vm[:, pl.ds(0, 2 * _LP)],
                         preferred_element_type=jnp.float32)
            sw = sf[:, :_LP] + sf[:, _LP:]
            z_ref[rows, :] = (g2_vm[rows, :][:, :_LP]
                              + dinv_vm[rows, :][:, :_LP] * sw)


def _encoder(adj, x, w1c, b1c, w2c, b2c):
    return pl.pallas_call(
        _enc_kernel,
        grid=(_NB + 1,),
        in_specs=[
            pl.BlockSpec((_BM, _N), lambda s: (jnp.minimum(s, _NB - 1), 0)),
            pl.BlockSpec((_BM, _D), lambda s: (jnp.minimum(s, _NB - 1), 0)),
            pl.BlockSpec((_D, _D), lambda s: (0, 0)),
            pl.BlockSpec((_D,), lambda s: (0,)),
            pl.BlockSpec((_H, 4 * _LP), lambda s: (0, 0)),
            pl.BlockSpec((4 * _LP,), lambda s: (0,)),
            pl.BlockSpec((_N, _H), lambda s: (0, 0)),
        ],
        out_specs=pl.BlockSpec((_N, _LP), lambda s: (0, 0)),
        out_shape=jax.ShapeDtypeStruct((_N, _LP), jnp.float32),
        scratch_shapes=[
            pltpu.VMEM((_N, _N), jnp.bfloat16),
            pltpu.VMEM((_N, 3 * _H), jnp.float32),
            pltpu.VMEM((_N, _H), jnp.float32),
            pltpu.VMEM((_N, 2 * _H), jnp.bfloat16),
            pltpu.VMEM((_N, 2 * _H), jnp.bfloat16),
            pltpu.VMEM((_N, 4 * _LP), jnp.float32),
        ],
        compiler_params=pltpu.CompilerParams(
            dimension_semantics=("arbitrary",),
        ),
    )(adj, x, w1c, b1c, w2c, b2c,
      jnp.ones((_N, _H), jnp.bfloat16))


def _zq_kernel(z_ref, mut_ref, wd_ref, bd_ref, wb_ref, q_ref, m_ref, hd_ref):
    z = z_ref[...]
    mut = mut_ref[...]
    z2 = jnp.sum(z * z, axis=1, keepdims=True)
    mu2 = jnp.sum(mut * mut, axis=0, keepdims=True)
    d2 = z2 - 2.0 * jnp.dot(z, mut, preferred_element_type=jnp.float32) + mu2
    qraw = 1.0 / (1.0 + d2)
    lane = jax.lax.broadcasted_iota(jnp.int32, qraw.shape, 1)
    qraw = jnp.where(lane < _NC, qraw, 0.0)
    q_ref[...] = qraw / jnp.sum(qraw, axis=1, keepdims=True)
    hd = jnp.dot(z, wd_ref[...], preferred_element_type=jnp.float32) + bd_ref[...][None, :]
    hd_ref[...] = hd
    m_ref[...] = jnp.dot(hd, wb_ref[...], preferred_element_type=jnp.float32)


def _zq(z_pad, mut, wd_p, bd, wb, bm=1024):
    n = z_pad.shape[0]
    return pl.pallas_call(
        _zq_kernel,
        grid=(n // bm,),
        in_specs=[
            pl.BlockSpec((bm, _LP), lambda i: (i, 0)),
            pl.BlockSpec((_LP, _LP), lambda i: (0, 0)),
            pl.BlockSpec((_LP, _ADJ), lambda i: (0, 0)),
            pl.BlockSpec((_ADJ,), lambda i: (0,)),
            pl.BlockSpec((_ADJ, _ADJ), lambda i: (0, 0)),
        ],
        out_specs=[
            pl.BlockSpec((bm, _LP), lambda i: (i, 0)),
            pl.BlockSpec((bm, _ADJ), lambda i: (i, 0)),
            pl.BlockSpec((bm, _ADJ), lambda i: (i, 0)),
        ],
        out_shape=[
            jax.ShapeDtypeStruct((n, _LP), jnp.float32),
            jax.ShapeDtypeStruct((n, _ADJ), jnp.float32),
            jax.ShapeDtypeStruct((n, _ADJ), jnp.float32),
        ],
    )(z_pad, mut, wd_p, bd, wb)


def _aout_kernel(m_ref, hd_ref, o_ref):
    prod = jax.lax.dot_general(
        m_ref[...], hd_ref[...], (((1,), (1,)), ((), ())),
        preferred_element_type=jnp.float32,
    )
    o_ref[...] = jax.nn.sigmoid(prod)


def _aout(m, hd, bm=1024):
    n = m.shape[0]
    return pl.pallas_call(
        _aout_kernel,
        grid=(n // bm, n // bm),
        in_specs=[
            pl.BlockSpec((bm, _ADJ), lambda i, j: (i, 0)),
            pl.BlockSpec((bm, _ADJ), lambda i, j: (j, 0)),
        ],
        out_specs=pl.BlockSpec((bm, bm), lambda i, j: (i, j)),
        out_shape=jax.ShapeDtypeStruct((n, n), jnp.float32),
        compiler_params=pltpu.CompilerParams(
            dimension_semantics=("parallel", "parallel"),
        ),
    )(m, hd)


def _decx_kernel(z_ref, wx1_ref, bx1_ref, wx2_ref, bx2_ref, wx3_ref, bx3_ref,
                 wpi_ref, bpi_ref, wdisp_ref, bdisp_ref, wmean_ref, bmean_ref,
                 pi_ref, disp_ref, mean_ref):
    f32 = jnp.float32
    hx = jnp.maximum(jnp.dot(z_ref[...], wx1_ref[...], preferred_element_type=f32)
                     + bx1_ref[...][None, :], 0.0)
    hx = jnp.maximum(jnp.dot(hx, wx2_ref[...], preferred_element_type=f32)
                     + bx2_ref[...][None, :], 0.0)
    hx = jnp.maximum(jnp.dot(hx, wx3_ref[...], preferred_element_type=f32)
                     + bx3_ref[...][None, :], 0.0)
    pi_ref[...] = jax.nn.sigmoid(
        jnp.dot(hx, wpi_ref[...], preferred_element_type=f32) + bpi_ref[...][None, :])
    t = jnp.dot(hx, wdisp_ref[...], preferred_element_type=f32) + bdisp_ref[...][None, :]
    sp = jnp.maximum(t, 0.0) + jnp.log1p(jnp.exp(-jnp.abs(t)))
    disp_ref[...] = jnp.clip(sp, 1e-4, 1e4)
    t = jnp.dot(hx, wmean_ref[...], preferred_element_type=f32) + bmean_ref[...][None, :]
    mean_ref[...] = jnp.clip(jnp.exp(t), 1e-5, 1e6)


def _decx(z_pad, wx1_p, bx1, wx2, bx2, wx3, bx3, wpi, bpi, wdisp, bdisp,
          wmean, bmean, bm=256):
    n = z_pad.shape[0]
    d0, d1, d2 = 128, 256, 512
    full = lambda a, b: pl.BlockSpec((a, b), lambda i: (0, 0))
    vec = lambda a: pl.BlockSpec((a,), lambda i: (0,))
    return pl.pallas_call(
        _decx_kernel,
        grid=(n // bm,),
        in_specs=[
            pl.BlockSpec((bm, _LP), lambda i: (i, 0)),
            full(_LP, d0), vec(d0), full(d0, d1), vec(d1), full(d1, d2), vec(d2),
            full(d2, _D), vec(_D), full(d2, _D), vec(_D), full(d2, _D), vec(_D),
        ],
        out_specs=[
            pl.BlockSpec((bm, _D), lambda i: (i, 0)),
            pl.BlockSpec((bm, _D), lambda i: (i, 0)),
            pl.BlockSpec((bm, _D), lambda i: (i, 0)),
        ],
        out_shape=[
            jax.ShapeDtypeStruct((n, _D), jnp.float32),
            jax.ShapeDtypeStruct((n, _D), jnp.float32),
            jax.ShapeDtypeStruct((n, _D), jnp.float32),
        ],
    )(z_pad, wx1_p, bx1, wx2, bx2, wx3, bx3, wpi, bpi, wdisp, bdisp, wmean, bmean)


def kernel(X, adj_n, W1, b1, W2, b2, mu, Wd, bd, Wb, Wx1, bx1, Wx2, bx2,
           Wx3, bx3, Wpi, bpi, Wdisp, bdisp, Wmean, bmean):
    f32 = jnp.float32

    w1c = W1.reshape(_K + 1, _D, _H).transpose(1, 0, 2).reshape(_D, (_K + 1) * _H)
    b1c = jnp.concatenate([b1, jnp.zeros((_K * _H,), f32)])
    w2c = jnp.zeros((_H, 4 * _LP), f32)
    for k in range(_K + 1):
        w2c = w2c.at[:, k * _LP:k * _LP + _L].set(W2[k * _H:(k + 1) * _H])
    b2c = jnp.zeros((4 * _LP,), f32).at[:_L].set(b2)
    mut = jnp.zeros((_LP, _LP), f32).at[:_L, :_NC].set(mu.T)
    wd_p = jnp.zeros((_LP, _ADJ), f32).at[:_L].set(Wd)
    wx1_p = jnp.zeros((_LP, 128), f32).at[:_L].set(Wx1)

    z_pad = _encoder(adj_n, X, w1c, b1c, w2c, b2c)

    q_out, m, hd = _zq(z_pad, mut, wd_p, bd, Wb)
    A_out = _aout(m, hd)

    pi, disp, mean = _decx(z_pad, wx1_p, bx1, Wx2, bx2, Wx3, bx3,
                           Wpi, bpi, Wdisp, bdisp, Wmean, bmean)

    return z_pad[:, :_L], q_out[:, :_NC], A_out, pi, disp, mean


# speedup vs baseline: 1.0060x; 1.0060x over previous
import functools

import jax
import jax.numpy as jnp
from jax.experimental import pallas as pl
from jax.experimental.pallas import tpu as pltpu

_N, _D, _H, _L = 4096, 512, 128, 15
_NC, _ADJ, _K = 15, 32, 3
_LP = 16


def _hilo(u, w):
    hi = u.astype(jnp.bfloat16)
    lo = (u - hi.astype(jnp.float32)).astype(jnp.bfloat16)
    return jnp.concatenate([hi, lo], axis=1)


_NB = 32
_BM = _N // _NB


def _enc_kernel(a_ref, x_ref, w1c_ref, b1c_ref, w2c_ref, b2c_ref,
                z_ref,
                b_vm, g_vm, dinv_vm, ua_vm, ub_vm, u2_vm, g2_vm):
    s = pl.program_id(0)

    @pl.when(s < _NB)
    def _scan():
        a = a_ref[...]
        mask = (a != 0.0).astype(jnp.float32)
        rows = pl.ds(jnp.minimum(s, _NB - 1) * _BM, _BM)
        b_vm[rows, :] = mask.astype(jnp.bfloat16)
        deg = jnp.sum(mask, axis=1, keepdims=True)
        dinv = jnp.where(deg > 0.0, jax.lax.rsqrt(deg), 0.0)
        dinvb = jnp.broadcast_to(dinv, (_BM, _H))
        dinv_vm[rows, :] = dinvb
        g = (jnp.dot(x_ref[...], w1c_ref[...], preferred_element_type=jnp.float32)
             + b1c_ref[...][None, :])
        g_vm[rows, :] = g[:, :3 * _H]
        ua_vm[rows, :] = (dinvb * g[:, 3 * _H:]).astype(jnp.bfloat16)

    @pl.when(s == _NB)
    def _hops():
        cm = 1024
        nch = _N // cm
        for k, src, dst in ((2, ua_vm, ub_vm), (1, ub_vm, ua_vm)):
            for c in range(nch):
                rows = pl.ds(c * cm, cm)
                sw = jnp.dot(b_vm[rows, :], src[...],
                             preferred_element_type=jnp.float32)
                dinv = dinv_vm[rows, :]
                p = g_vm[rows, :][:, k * _H:(k + 1) * _H] + dinv * sw
                dst[rows, :] = (dinv * p).astype(jnp.bfloat16)
        for c in range(nch):
            rows = pl.ds(c * cm, cm)
            sw = jnp.dot(b_vm[rows, :], ua_vm[...],
                         preferred_element_type=jnp.float32)
            dinv = dinv_vm[rows, :]
            h = jnp.maximum(g_vm[rows, :][:, :_H] + dinv * sw, 0.0)
            g2 = (jnp.dot(h, w2c_ref[...], preferred_element_type=jnp.float32)
                  + b2c_ref[...][None, :])
            g2_vm[rows, :] = g2
            u2_vm[rows, pl.ds(0, 2 * _LP)] = _hilo(
                dinv[:, :_LP] * g2[:, 3 * _LP:], _LP)
        for k, so, do in ((2, 0, 2 * _LP), (1, 2 * _LP, 0)):
            for c in range(nch):
                rows = pl.ds(c * cm, cm)
                sf = jnp.dot(b_vm[rows, :], u2_vm[:, pl.ds(so, 2 * _LP)],
                             preferred_element_type=jnp.float32)
                sw = sf[:, :_LP] + sf[:, _LP:]
                dinv = dinv_vm[rows, :][:, :_LP]
                p = g2_vm[rows, :][:, k * _LP:(k + 1) * _LP] + dinv * sw
                u2_vm[rows, pl.ds(do, 2 * _LP)] = _hilo(dinv * p, _LP)
        for c in range(nch):
            rows = pl.ds(c * cm, cm)
            sf = jnp.dot(b_vm[rows, :], u2_vm[:, pl.ds(0, 2 * _LP)],
                         preferred_element_type=jnp.float32)
            sw = sf[:, :_LP] + sf[:, _LP:]
            z_ref[rows, :] = (g2_vm[rows, :][:, :_LP]
                              + dinv_vm[rows, :][:, :_LP] * sw)


def _encoder(adj, x, w1c, b1c, w2c, b2c):
    return pl.pallas_call(
        _enc_kernel,
        grid=(_NB + 1,),
        in_specs=[
            pl.BlockSpec((_BM, _N), lambda s: (jnp.minimum(s, _NB - 1), 0)),
            pl.BlockSpec((_BM, _D), lambda s: (jnp.minimum(s, _NB - 1), 0)),
            pl.BlockSpec((_D, _D), lambda s: (0, 0)),
            pl.BlockSpec((_D,), lambda s: (0,)),
            pl.BlockSpec((_H, 4 * _LP), lambda s: (0, 0)),
            pl.BlockSpec((4 * _LP,), lambda s: (0,)),
        ],
        out_specs=pl.BlockSpec((_N, _LP), lambda s: (0, 0)),
        out_shape=jax.ShapeDtypeStruct((_N, _LP), jnp.float32),
        scratch_shapes=[
            pltpu.VMEM((_N, _N), jnp.bfloat16),
            pltpu.VMEM((_N, 3 * _H), jnp.float32),
            pltpu.VMEM((_N, _H), jnp.float32),
            pltpu.VMEM((_N, _H), jnp.bfloat16),
            pltpu.VMEM((_N, _H), jnp.bfloat16),
            pltpu.VMEM((_N, 4 * _LP), jnp.bfloat16),
            pltpu.VMEM((_N, 4 * _LP), jnp.float32),
        ],
        compiler_params=pltpu.CompilerParams(
            dimension_semantics=("arbitrary",),
        ),
    )(adj, x, w1c, b1c, w2c, b2c)


def _zq_kernel(z_ref, mut_ref, wd_ref, bd_ref, wb_ref, q_ref, m_ref, hd_ref):
    z = z_ref[...]
    mut = mut_ref[...]
    z2 = jnp.sum(z * z, axis=1, keepdims=True)
    mu2 = jnp.sum(mut * mut, axis=0, keepdims=True)
    d2 = z2 - 2.0 * jnp.dot(z, mut, preferred_element_type=jnp.float32) + mu2
    qraw = 1.0 / (1.0 + d2)
    lane = jax.lax.broadcasted_iota(jnp.int32, qraw.shape, 1)
    qraw = jnp.where(lane < _NC, qraw, 0.0)
    q_ref[...] = qraw / jnp.sum(qraw, axis=1, keepdims=True)
    hd = jnp.dot(z, wd_ref[...], preferred_element_type=jnp.float32) + bd_ref[...][None, :]
    hd_ref[...] = hd
    m_ref[...] = jnp.dot(hd, wb_ref[...], preferred_element_type=jnp.float32)


def _zq(z_pad, mut, wd_p, bd, wb, bm=1024):
    n = z_pad.shape[0]
    return pl.pallas_call(
        _zq_kernel,
        grid=(n // bm,),
        in_specs=[
            pl.BlockSpec((bm, _LP), lambda i: (i, 0)),
            pl.BlockSpec((_LP, _LP), lambda i: (0, 0)),
            pl.BlockSpec((_LP, _ADJ), lambda i: (0, 0)),
            pl.BlockSpec((_ADJ,), lambda i: (0,)),
            pl.BlockSpec((_ADJ, _ADJ), lambda i: (0, 0)),
        ],
        out_specs=[
            pl.BlockSpec((bm, _LP), lambda i: (i, 0)),
            pl.BlockSpec((bm, _ADJ), lambda i: (i, 0)),
            pl.BlockSpec((bm, _ADJ), lambda i: (i, 0)),
        ],
        out_shape=[
            jax.ShapeDtypeStruct((n, _LP), jnp.float32),
            jax.ShapeDtypeStruct((n, _ADJ), jnp.float32),
            jax.ShapeDtypeStruct((n, _ADJ), jnp.float32),
        ],
    )(z_pad, mut, wd_p, bd, wb)


def _aout_kernel(m_ref, hd_ref, o_ref):
    prod = jax.lax.dot_general(
        m_ref[...], hd_ref[...], (((1,), (1,)), ((), ())),
        preferred_element_type=jnp.float32,
    )
    o_ref[...] = jax.nn.sigmoid(prod)


def _aout(m, hd, bm=1024):
    n = m.shape[0]
    return pl.pallas_call(
        _aout_kernel,
        grid=(n // bm, n // bm),
        in_specs=[
            pl.BlockSpec((bm, _ADJ), lambda i, j: (i, 0)),
            pl.BlockSpec((bm, _ADJ), lambda i, j: (j, 0)),
        ],
        out_specs=pl.BlockSpec((bm, bm), lambda i, j: (i, j)),
        out_shape=jax.ShapeDtypeStruct((n, n), jnp.float32),
        compiler_params=pltpu.CompilerParams(
            dimension_semantics=("parallel", "parallel"),
        ),
    )(m, hd)


def _decx_kernel(z_ref, wx1_ref, bx1_ref, wx2_ref, bx2_ref, wx3_ref, bx3_ref,
                 wpi_ref, bpi_ref, wdisp_ref, bdisp_ref, wmean_ref, bmean_ref,
                 pi_ref, disp_ref, mean_ref):
    f32 = jnp.float32
    hx = jnp.maximum(jnp.dot(z_ref[...], wx1_ref[...], preferred_element_type=f32)
                     + bx1_ref[...][None, :], 0.0)
    hx = jnp.maximum(jnp.dot(hx, wx2_ref[...], preferred_element_type=f32)
                     + bx2_ref[...][None, :], 0.0)
    hx = jnp.maximum(jnp.dot(hx, wx3_ref[...], preferred_element_type=f32)
                     + bx3_ref[...][None, :], 0.0)
    pi_ref[...] = jax.nn.sigmoid(
        jnp.dot(hx, wpi_ref[...], preferred_element_type=f32) + bpi_ref[...][None, :])
    t = jnp.dot(hx, wdisp_ref[...], preferred_element_type=f32) + bdisp_ref[...][None, :]
    sp = jnp.maximum(t, 0.0) + jnp.log1p(jnp.exp(-jnp.abs(t)))
    disp_ref[...] = jnp.clip(sp, 1e-4, 1e4)
    t = jnp.dot(hx, wmean_ref[...], preferred_element_type=f32) + bmean_ref[...][None, :]
    mean_ref[...] = jnp.clip(jnp.exp(t), 1e-5, 1e6)


def _decx(z_pad, wx1_p, bx1, wx2, bx2, wx3, bx3, wpi, bpi, wdisp, bdisp,
          wmean, bmean, bm=256):
    n = z_pad.shape[0]
    d0, d1, d2 = 128, 256, 512
    full = lambda a, b: pl.BlockSpec((a, b), lambda i: (0, 0))
    vec = lambda a: pl.BlockSpec((a,), lambda i: (0,))
    return pl.pallas_call(
        _decx_kernel,
        grid=(n // bm,),
        in_specs=[
            pl.BlockSpec((bm, _LP), lambda i: (i, 0)),
            full(_LP, d0), vec(d0), full(d0, d1), vec(d1), full(d1, d2), vec(d2),
            full(d2, _D), vec(_D), full(d2, _D), vec(_D), full(d2, _D), vec(_D),
        ],
        out_specs=[
            pl.BlockSpec((bm, _D), lambda i: (i, 0)),
            pl.BlockSpec((bm, _D), lambda i: (i, 0)),
            pl.BlockSpec((bm, _D), lambda i: (i, 0)),
        ],
        out_shape=[
            jax.ShapeDtypeStruct((n, _D), jnp.float32),
            jax.ShapeDtypeStruct((n, _D), jnp.float32),
            jax.ShapeDtypeStruct((n, _D), jnp.float32),
        ],
    )(z_pad, wx1_p, bx1, wx2, bx2, wx3, bx3, wpi, bpi, wdisp, bdisp, wmean, bmean)


def kernel(X, adj_n, W1, b1, W2, b2, mu, Wd, bd, Wb, Wx1, bx1, Wx2, bx2,
           Wx3, bx3, Wpi, bpi, Wdisp, bdisp, Wmean, bmean):
    f32 = jnp.float32

    w1c = W1.reshape(_K + 1, _D, _H).transpose(1, 0, 2).reshape(_D, (_K + 1) * _H)
    b1c = jnp.concatenate([b1, jnp.zeros((_K * _H,), f32)])
    w2c = jnp.zeros((_H, 4 * _LP), f32)
    for k in range(_K + 1):
        w2c = w2c.at[:, k * _LP:k * _LP + _L].set(W2[k * _H:(k + 1) * _H])
    b2c = jnp.zeros((4 * _LP,), f32).at[:_L].set(b2)
    mut = jnp.zeros((_LP, _LP), f32).at[:_L, :_NC].set(mu.T)
    wd_p = jnp.zeros((_LP, _ADJ), f32).at[:_L].set(Wd)
    wx1_p = jnp.zeros((_LP, 128), f32).at[:_L].set(Wx1)

    z_pad = _encoder(adj_n, X, w1c, b1c, w2c, b2c)

    q_out, m, hd = _zq(z_pad, mut, wd_p, bd, Wb)
    A_out = _aout(m, hd)

    pi, disp, mean = _decx(z_pad, wx1_p, bx1, Wx2, bx2, Wx3, bx3,
                           Wpi, bpi, Wdisp, bdisp, Wmean, bmean)

    return z_pad[:, :_L], q_out[:, :_NC], A_out, pi, disp, mean


# speedup vs baseline: 1.0120x; 1.0060x over previous
import functools

import jax
import jax.numpy as jnp
from jax.experimental import pallas as pl
from jax.experimental.pallas import tpu as pltpu

_N, _D, _H, _L = 4096, 512, 128, 15
_NC, _ADJ, _K = 15, 32, 3
_LP = 16


def _hilo(u, w):
    hi = u.astype(jnp.bfloat16)
    lo = (u - hi.astype(jnp.float32)).astype(jnp.bfloat16)
    return jnp.concatenate([hi, lo], axis=1)


_NB = 32
_BM = _N // _NB


def _enc_kernel(a_ref, a2_ref, x_ref, w1c_ref, b1c_ref, w2c_ref, b2c_ref,
                z_ref,
                b_vm, g_vm, dinv_vm, ua_vm, ub_vm, u2_vm, g2_vm):
    s = pl.program_id(0)

    @pl.when(s < _NB)
    def _scan():
        rows = pl.ds(jnp.minimum(s, _NB - 1) * _BM, _BM)
        mask = (a_ref[...] != 0.0).astype(jnp.float32)
        mask2 = (a2_ref[...] != 0.0).astype(jnp.float32)
        b_vm[rows, pl.ds(0, _N // 2)] = mask.astype(jnp.bfloat16)
        b_vm[rows, pl.ds(_N // 2, _N // 2)] = mask2.astype(jnp.bfloat16)
        deg = (jnp.sum(mask, axis=1, keepdims=True)
               + jnp.sum(mask2, axis=1, keepdims=True))
        dinv = jnp.where(deg > 0.0, jax.lax.rsqrt(deg), 0.0)
        dinvb = jnp.broadcast_to(dinv, (_BM, _H))
        dinv_vm[rows, :] = dinvb
        g = (jnp.dot(x_ref[...], w1c_ref[...], preferred_element_type=jnp.float32)
             + b1c_ref[...][None, :])
        g_vm[rows, :] = g[:, :3 * _H]
        ua_vm[rows, :] = (dinvb * g[:, 3 * _H:]).astype(jnp.bfloat16)

    @pl.when(s == _NB)
    def _hops():
        cm = 1024
        nch = _N // cm
        for k, src, dst in ((2, ua_vm, ub_vm), (1, ub_vm, ua_vm)):
            for c in range(nch):
                rows = pl.ds(c * cm, cm)
                sw = jnp.dot(b_vm[rows, :], src[...],
                             preferred_element_type=jnp.float32)
                dinv = dinv_vm[rows, :]
                p = g_vm[rows, :][:, k * _H:(k + 1) * _H] + dinv * sw
                dst[rows, :] = (dinv * p).astype(jnp.bfloat16)
        for c in range(nch):
            rows = pl.ds(c * cm, cm)
            sw = jnp.dot(b_vm[rows, :], ua_vm[...],
                         preferred_element_type=jnp.float32)
            dinv = dinv_vm[rows, :]
            h = jnp.maximum(g_vm[rows, :][:, :_H] + dinv * sw, 0.0)
            g2 = (jnp.dot(h, w2c_ref[...], preferred_element_type=jnp.float32)
                  + b2c_ref[...][None, :])
            g2_vm[rows, :] = g2
            u2_vm[rows, pl.ds(0, 2 * _LP)] = _hilo(
                dinv[:, :_LP] * g2[:, 3 * _LP:], _LP)
        for k, so, do in ((2, 0, 2 * _LP), (1, 2 * _LP, 0)):
            for c in range(nch):
                rows = pl.ds(c * cm, cm)
                sf = jnp.dot(b_vm[rows, :], u2_vm[:, pl.ds(so, 2 * _LP)],
                             preferred_element_type=jnp.float32)
                sw = sf[:, :_LP] + sf[:, _LP:]
                dinv = dinv_vm[rows, :][:, :_LP]
                p = g2_vm[rows, :][:, k * _LP:(k + 1) * _LP] + dinv * sw
                u2_vm[rows, pl.ds(do, 2 * _LP)] = _hilo(dinv * p, _LP)
        for c in range(nch):
            rows = pl.ds(c * cm, cm)
            sf = jnp.dot(b_vm[rows, :], u2_vm[:, pl.ds(0, 2 * _LP)],
                         preferred_element_type=jnp.float32)
            sw = sf[:, :_LP] + sf[:, _LP:]
            z_ref[rows, :] = (g2_vm[rows, :][:, :_LP]
                              + dinv_vm[rows, :][:, :_LP] * sw)


def _encoder(adj, x, w1c, b1c, w2c, b2c):
    return pl.pallas_call(
        _enc_kernel,
        grid=(_NB + 1,),
        in_specs=[
            pl.BlockSpec((_BM, _N // 2), lambda s: (jnp.minimum(s, _NB - 1), 0)),
            pl.BlockSpec((_BM, _N // 2), lambda s: (jnp.minimum(s, _NB - 1), 1)),
            pl.BlockSpec((_BM, _D), lambda s: (jnp.minimum(s, _NB - 1), 0)),
            pl.BlockSpec((_D, _D), lambda s: (0, 0)),
            pl.BlockSpec((_D,), lambda s: (0,)),
            pl.BlockSpec((_H, 4 * _LP), lambda s: (0, 0)),
            pl.BlockSpec((4 * _LP,), lambda s: (0,)),
        ],
        out_specs=pl.BlockSpec((_N, _LP), lambda s: (0, 0)),
        out_shape=jax.ShapeDtypeStruct((_N, _LP), jnp.float32),
        scratch_shapes=[
            pltpu.VMEM((_N, _N), jnp.bfloat16),
            pltpu.VMEM((_N, 3 * _H), jnp.float32),
            pltpu.VMEM((_N, _H), jnp.float32),
            pltpu.VMEM((_N, _H), jnp.bfloat16),
            pltpu.VMEM((_N, _H), jnp.bfloat16),
            pltpu.VMEM((_N, 4 * _LP), jnp.bfloat16),
            pltpu.VMEM((_N, 4 * _LP), jnp.float32),
        ],
        compiler_params=pltpu.CompilerParams(
            dimension_semantics=("arbitrary",),
        ),
    )(adj, adj, x, w1c, b1c, w2c, b2c)


def _zq_kernel(z_ref, mut_ref, wd_ref, bd_ref, wb_ref, q_ref, m_ref, hd_ref):
    z = z_ref[...]
    mut = mut_ref[...]
    z2 = jnp.sum(z * z, axis=1, keepdims=True)
    mu2 = jnp.sum(mut * mut, axis=0, keepdims=True)
    d2 = z2 - 2.0 * jnp.dot(z, mut, preferred_element_type=jnp.float32) + mu2
    qraw = 1.0 / (1.0 + d2)
    lane = jax.lax.broadcasted_iota(jnp.int32, qraw.shape, 1)
    qraw = jnp.where(lane < _NC, qraw, 0.0)
    q_ref[...] = qraw / jnp.sum(qraw, axis=1, keepdims=True)
    hd = jnp.dot(z, wd_ref[...], preferred_element_type=jnp.float32) + bd_ref[...][None, :]
    hd_ref[...] = hd
    m_ref[...] = jnp.dot(hd, wb_ref[...], preferred_element_type=jnp.float32)


def _zq(z_pad, mut, wd_p, bd, wb, bm=1024):
    n = z_pad.shape[0]
    return pl.pallas_call(
        _zq_kernel,
        grid=(n // bm,),
        in_specs=[
            pl.BlockSpec((bm, _LP), lambda i: (i, 0)),
            pl.BlockSpec((_LP, _LP), lambda i: (0, 0)),
            pl.BlockSpec((_LP, _ADJ), lambda i: (0, 0)),
            pl.BlockSpec((_ADJ,), lambda i: (0,)),
            pl.BlockSpec((_ADJ, _ADJ), lambda i: (0, 0)),
        ],
        out_specs=[
            pl.BlockSpec((bm, _LP), lambda i: (i, 0)),
            pl.BlockSpec((bm, _ADJ), lambda i: (i, 0)),
            pl.BlockSpec((bm, _ADJ), lambda i: (i, 0)),
        ],
        out_shape=[
            jax.ShapeDtypeStruct((n, _LP), jnp.float32),
            jax.ShapeDtypeStruct((n, _ADJ), jnp.float32),
            jax.ShapeDtypeStruct((n, _ADJ), jnp.float32),
        ],
    )(z_pad, mut, wd_p, bd, wb)


def _aout_kernel(m_ref, hd_ref, o_ref):
    prod = jax.lax.dot_general(
        m_ref[...], hd_ref[...], (((1,), (1,)), ((), ())),
        preferred_element_type=jnp.float32,
    )
    o_ref[...] = jax.nn.sigmoid(prod)


def _aout(m, hd, bm=1024):
    n = m.shape[0]
    return pl.pallas_call(
        _aout_kernel,
        grid=(n // bm, n // bm),
        in_specs=[
            pl.BlockSpec((bm, _ADJ), lambda i, j: (i, 0)),
            pl.BlockSpec((bm, _ADJ), lambda i, j: (j, 0)),
        ],
        out_specs=pl.BlockSpec((bm, bm), lambda i, j: (i, j)),
        out_shape=jax.ShapeDtypeStruct((n, n), jnp.float32),
        compiler_params=pltpu.CompilerParams(
            dimension_semantics=("parallel", "parallel"),
        ),
    )(m, hd)


def _decx_kernel(z_ref, wx1_ref, bx1_ref, wx2_ref, bx2_ref, wx3_ref, bx3_ref,
                 wpi_ref, bpi_ref, wdisp_ref, bdisp_ref, wmean_ref, bmean_ref,
                 pi_ref, disp_ref, mean_ref):
    f32 = jnp.float32
    hx = jnp.maximum(jnp.dot(z_ref[...], wx1_ref[...], preferred_element_type=f32)
                     + bx1_ref[...][None, :], 0.0)
    hx = jnp.maximum(jnp.dot(hx, wx2_ref[...], preferred_element_type=f32)
                     + bx2_ref[...][None, :], 0.0)
    hx = jnp.maximum(jnp.dot(hx, wx3_ref[...], preferred_element_type=f32)
                     + bx3_ref[...][None, :], 0.0)
    pi_ref[...] = jax.nn.sigmoid(
        jnp.dot(hx, wpi_ref[...], preferred_element_type=f32) + bpi_ref[...][None, :])
    t = jnp.dot(hx, wdisp_ref[...], preferred_element_type=f32) + bdisp_ref[...][None, :]
    sp = jnp.maximum(t, 0.0) + jnp.log1p(jnp.exp(-jnp.abs(t)))
    disp_ref[...] = jnp.clip(sp, 1e-4, 1e4)
    t = jnp.dot(hx, wmean_ref[...], preferred_element_type=f32) + bmean_ref[...][None, :]
    mean_ref[...] = jnp.clip(jnp.exp(t), 1e-5, 1e6)


def _decx(z_pad, wx1_p, bx1, wx2, bx2, wx3, bx3, wpi, bpi, wdisp, bdisp,
          wmean, bmean, bm=256):
    n = z_pad.shape[0]
    d0, d1, d2 = 128, 256, 512
    full = lambda a, b: pl.BlockSpec((a, b), lambda i: (0, 0))
    vec = lambda a: pl.BlockSpec((a,), lambda i: (0,))
    return pl.pallas_call(
        _decx_kernel,
        grid=(n // bm,),
        in_specs=[
            pl.BlockSpec((bm, _LP), lambda i: (i, 0)),
            full(_LP, d0), vec(d0), full(d0, d1), vec(d1), full(d1, d2), vec(d2),
            full(d2, _D), vec(_D), full(d2, _D), vec(_D), full(d2, _D), vec(_D),
        ],
        out_specs=[
            pl.BlockSpec((bm, _D), lambda i: (i, 0)),
            pl.BlockSpec((bm, _D), lambda i: (i, 0)),
            pl.BlockSpec((bm, _D), lambda i: (i, 0)),
        ],
        out_shape=[
            jax.ShapeDtypeStruct((n, _D), jnp.float32),
            jax.ShapeDtypeStruct((n, _D), jnp.float32),
            jax.ShapeDtypeStruct((n, _D), jnp.float32),
        ],
    )(z_pad, wx1_p, bx1, wx2, bx2, wx3, bx3, wpi, bpi, wdisp, bdisp, wmean, bmean)


def kernel(X, adj_n, W1, b1, W2, b2, mu, Wd, bd, Wb, Wx1, bx1, Wx2, bx2,
           Wx3, bx3, Wpi, bpi, Wdisp, bdisp, Wmean, bmean):
    f32 = jnp.float32

    w1c = W1.reshape(_K + 1, _D, _H).transpose(1, 0, 2).reshape(_D, (_K + 1) * _H)
    b1c = jnp.concatenate([b1, jnp.zeros((_K * _H,), f32)])
    w2c = jnp.zeros((_H, 4 * _LP), f32)
    for k in range(_K + 1):
        w2c = w2c.at[:, k * _LP:k * _LP + _L].set(W2[k * _H:(k + 1) * _H])
    b2c = jnp.zeros((4 * _LP,), f32).at[:_L].set(b2)
    mut = jnp.zeros((_LP, _LP), f32).at[:_L, :_NC].set(mu.T)
    wd_p = jnp.zeros((_LP, _ADJ), f32).at[:_L].set(Wd)
    wx1_p = jnp.zeros((_LP, 128), f32).at[:_L].set(Wx1)

    z_pad = _encoder(adj_n, X, w1c, b1c, w2c, b2c)

    q_out, m, hd = _zq(z_pad, mut, wd_p, bd, Wb)
    A_out = _aout(m, hd)

    pi, disp, mean = _decx(z_pad, wx1_p, bx1, Wx2, bx2, Wx3, bx3,
                           Wpi, bpi, Wdisp, bdisp, Wmean, bmean)

    return z_pad[:, :_L], q_out[:, :_NC], A_out, pi, disp, mean


# speedup vs baseline: 1.0224x; 1.0103x over previous
import functools

import jax
import jax.numpy as jnp
from jax.experimental import pallas as pl
from jax.experimental.pallas import tpu as pltpu

_N, _D, _H, _L = 4096, 512, 128, 15
_NC, _ADJ, _K = 15, 32, 3
_LP = 16


def _hilo(u, w):
    hi = u.astype(jnp.bfloat16)
    lo = (u - hi.astype(jnp.float32)).astype(jnp.bfloat16)
    return jnp.concatenate([hi, lo], axis=1)


_NB = 32
_BM = _N // _NB


def _enc_kernel(a_ref, a2_ref, x_ref, w1c_ref, b1c_ref, w2c_ref, b2c_ref,
                z_ref,
                b_vm, g_vm, dinv_vm, ua_vm, ub_vm, u2_vm, g2_vm):
    s = pl.program_id(0)

    @pl.when(s < _NB)
    def _scan():
        rows = pl.ds(jnp.minimum(s, _NB - 1) * _BM, _BM)
        mask = (a_ref[...] != 0.0).astype(jnp.float32)
        mask2 = (a2_ref[...] != 0.0).astype(jnp.float32)
        b_vm[rows, pl.ds(0, _N // 2)] = mask.astype(jnp.bfloat16)
        b_vm[rows, pl.ds(_N // 2, _N // 2)] = mask2.astype(jnp.bfloat16)
        deg = (jnp.sum(mask, axis=1, keepdims=True)
               + jnp.sum(mask2, axis=1, keepdims=True))
        dinv = jnp.where(deg > 0.0, jax.lax.rsqrt(deg), 0.0)
        dinvb = jnp.broadcast_to(dinv, (_BM, _H))
        dinv_vm[rows, :] = dinvb
        g = (jnp.dot(x_ref[...], w1c_ref[...], preferred_element_type=jnp.float32)
             + b1c_ref[...][None, :])
        g_vm[rows, :] = g[:, :3 * _H]
        ua_vm[rows, :] = (dinvb * g[:, 3 * _H:]).astype(jnp.bfloat16)

    @pl.when(s == _NB)
    def _hops():
        cm = 1024
        nch = _N // cm
        for k, src, dst in ((2, ua_vm, ub_vm), (1, ub_vm, ua_vm)):
            for c in range(nch):
                rows = pl.ds(c * cm, cm)
                sw = jnp.dot(b_vm[rows, :], src[...],
                             preferred_element_type=jnp.float32)
                dinv = dinv_vm[rows, :]
                p = g_vm[rows, :][:, k * _H:(k + 1) * _H] + dinv * sw
                dst[rows, :] = (dinv * p).astype(jnp.bfloat16)
        for c in range(nch):
            rows = pl.ds(c * cm, cm)
            sw = jnp.dot(b_vm[rows, :], ua_vm[...],
                         preferred_element_type=jnp.float32)
            dinv = dinv_vm[rows, :]
            h = jnp.maximum(g_vm[rows, :][:, :_H] + dinv * sw, 0.0)
            g2 = (jnp.dot(h, w2c_ref[...], preferred_element_type=jnp.float32)
                  + b2c_ref[...][None, :])
            g2_vm[rows, :] = g2
            u2_vm[rows, pl.ds(0, 2 * _LP)] = _hilo(
                dinv[:, :_LP] * g2[:, 3 * _LP:], _LP)
        for k, so, do in ((2, 0, 2 * _LP), (1, 2 * _LP, 0)):
            for c in range(nch):
                rows = pl.ds(c * cm, cm)
                sf = jnp.dot(b_vm[rows, :], u2_vm[:, pl.ds(so, 2 * _LP)],
                             preferred_element_type=jnp.float32)
                sw = sf[:, :_LP] + sf[:, _LP:]
                dinv = dinv_vm[rows, :][:, :_LP]
                p = g2_vm[rows, :][:, k * _LP:(k + 1) * _LP] + dinv * sw
                u2_vm[rows, pl.ds(do, 2 * _LP)] = _hilo(dinv * p, _LP)
        for c in range(nch):
            rows = pl.ds(c * cm, cm)
            sf = jnp.dot(b_vm[rows, :], u2_vm[:, pl.ds(0, 2 * _LP)],
                         preferred_element_type=jnp.float32)
            sw = sf[:, :_LP] + sf[:, _LP:]
            z_ref[rows, :] = (g2_vm[rows, :][:, :_LP]
                              + dinv_vm[rows, :][:, :_LP] * sw)


def _encoder(adj, x, w1c, b1c, w2c, b2c):
    return pl.pallas_call(
        _enc_kernel,
        grid=(_NB + 1,),
        in_specs=[
            pl.BlockSpec((_BM, _N // 2), lambda s: (jnp.minimum(s, _NB - 1), 0)),
            pl.BlockSpec((_BM, _N // 2), lambda s: (jnp.minimum(s, _NB - 1), 1)),
            pl.BlockSpec((_BM, _D), lambda s: (jnp.minimum(s, _NB - 1), 0)),
            pl.BlockSpec((_D, _D), lambda s: (0, 0)),
            pl.BlockSpec((_D,), lambda s: (0,)),
            pl.BlockSpec((_H, 4 * _LP), lambda s: (0, 0)),
            pl.BlockSpec((4 * _LP,), lambda s: (0,)),
        ],
        out_specs=pl.BlockSpec((_N, _LP), lambda s: (0, 0)),
        out_shape=jax.ShapeDtypeStruct((_N, _LP), jnp.float32),
        scratch_shapes=[
            pltpu.VMEM((_N, _N), jnp.bfloat16),
            pltpu.VMEM((_N, 3 * _H), jnp.float32),
            pltpu.VMEM((_N, _H), jnp.float32),
            pltpu.VMEM((_N, _H), jnp.bfloat16),
            pltpu.VMEM((_N, _H), jnp.bfloat16),
            pltpu.VMEM((_N, 4 * _LP), jnp.bfloat16),
            pltpu.VMEM((_N, 4 * _LP), jnp.float32),
        ],
        compiler_params=pltpu.CompilerParams(
            dimension_semantics=("arbitrary",),
        ),
    )(adj, adj, x, w1c, b1c, w2c, b2c)


def _zq_kernel(z_ref, mut_ref, wd_ref, bd_ref, wb_ref, q_ref, m_ref, hd_ref):
    z = z_ref[...]
    mut = mut_ref[...]
    z2 = jnp.sum(z * z, axis=1, keepdims=True)
    mu2 = jnp.sum(mut * mut, axis=0, keepdims=True)
    d2 = z2 - 2.0 * jnp.dot(z, mut, preferred_element_type=jnp.float32) + mu2
    qraw = 1.0 / (1.0 + d2)
    lane = jax.lax.broadcasted_iota(jnp.int32, qraw.shape, 1)
    qraw = jnp.where(lane < _NC, qraw, 0.0)
    q_ref[...] = qraw / jnp.sum(qraw, axis=1, keepdims=True)
    hd = jnp.dot(z, wd_ref[...], preferred_element_type=jnp.float32) + bd_ref[...][None, :]
    hd_ref[...] = hd
    m_ref[...] = jnp.dot(hd, wb_ref[...], preferred_element_type=jnp.float32)


def _zq(z_pad, mut, wd_p, bd, wb, bm=1024):
    n = z_pad.shape[0]
    return pl.pallas_call(
        _zq_kernel,
        grid=(n // bm,),
        in_specs=[
            pl.BlockSpec((bm, _LP), lambda i: (i, 0)),
            pl.BlockSpec((_LP, _LP), lambda i: (0, 0)),
            pl.BlockSpec((_LP, _ADJ), lambda i: (0, 0)),
            pl.BlockSpec((_ADJ,), lambda i: (0,)),
            pl.BlockSpec((_ADJ, _ADJ), lambda i: (0, 0)),
        ],
        out_specs=[
            pl.BlockSpec((bm, _LP), lambda i: (i, 0)),
            pl.BlockSpec((bm, _ADJ), lambda i: (i, 0)),
            pl.BlockSpec((bm, _ADJ), lambda i: (i, 0)),
        ],
        out_shape=[
            jax.ShapeDtypeStruct((n, _LP), jnp.float32),
            jax.ShapeDtypeStruct((n, _ADJ), jnp.float32),
            jax.ShapeDtypeStruct((n, _ADJ), jnp.float32),
        ],
    )(z_pad, mut, wd_p, bd, wb)


def _aout_kernel(m_ref, hd_ref, o_ref):
    prod = jax.lax.dot_general(
        m_ref[...], hd_ref[...], (((1,), (1,)), ((), ())),
        preferred_element_type=jnp.float32,
    )
    o_ref[...] = jax.nn.sigmoid(prod)


def _aout(m, hd, bm=2048):
    n = m.shape[0]
    return pl.pallas_call(
        _aout_kernel,
        grid=(n // bm, n // bm),
        in_specs=[
            pl.BlockSpec((bm, _ADJ), lambda i, j: (i, 0)),
            pl.BlockSpec((bm, _ADJ), lambda i, j: (j, 0)),
        ],
        out_specs=pl.BlockSpec((bm, bm), lambda i, j: (i, j)),
        out_shape=jax.ShapeDtypeStruct((n, n), jnp.float32),
        compiler_params=pltpu.CompilerParams(
            dimension_semantics=("parallel", "parallel"),
        ),
    )(m, hd)


def _decx_kernel(z_ref, wx1_ref, bx1_ref, wx2_ref, bx2_ref, wx3_ref, bx3_ref,
                 wpi_ref, bpi_ref, wdisp_ref, bdisp_ref, wmean_ref, bmean_ref,
                 pi_ref, disp_ref, mean_ref):
    f32 = jnp.float32
    hx = jnp.maximum(jnp.dot(z_ref[...], wx1_ref[...], preferred_element_type=f32)
                     + bx1_ref[...][None, :], 0.0)
    hx = jnp.maximum(jnp.dot(hx, wx2_ref[...], preferred_element_type=f32)
                     + bx2_ref[...][None, :], 0.0)
    hx = jnp.maximum(jnp.dot(hx, wx3_ref[...], preferred_element_type=f32)
                     + bx3_ref[...][None, :], 0.0)
    pi_ref[...] = jax.nn.sigmoid(
        jnp.dot(hx, wpi_ref[...], preferred_element_type=f32) + bpi_ref[...][None, :])
    t = jnp.dot(hx, wdisp_ref[...], preferred_element_type=f32) + bdisp_ref[...][None, :]
    sp = jnp.maximum(t, 0.0) + jnp.log1p(jnp.exp(-jnp.abs(t)))
    disp_ref[...] = jnp.clip(sp, 1e-4, 1e4)
    t = jnp.dot(hx, wmean_ref[...], preferred_element_type=f32) + bmean_ref[...][None, :]
    mean_ref[...] = jnp.clip(jnp.exp(t), 1e-5, 1e6)


def _decx(z_pad, wx1_p, bx1, wx2, bx2, wx3, bx3, wpi, bpi, wdisp, bdisp,
          wmean, bmean, bm=512):
    n = z_pad.shape[0]
    d0, d1, d2 = 128, 256, 512
    full = lambda a, b: pl.BlockSpec((a, b), lambda i: (0, 0))
    vec = lambda a: pl.BlockSpec((a,), lambda i: (0,))
    return pl.pallas_call(
        _decx_kernel,
        grid=(n // bm,),
        in_specs=[
            pl.BlockSpec((bm, _LP), lambda i: (i, 0)),
            full(_LP, d0), vec(d0), full(d0, d1), vec(d1), full(d1, d2), vec(d2),
            full(d2, _D), vec(_D), full(d2, _D), vec(_D), full(d2, _D), vec(_D),
        ],
        out_specs=[
            pl.BlockSpec((bm, _D), lambda i: (i, 0)),
            pl.BlockSpec((bm, _D), lambda i: (i, 0)),
            pl.BlockSpec((bm, _D), lambda i: (i, 0)),
        ],
        out_shape=[
            jax.ShapeDtypeStruct((n, _D), jnp.float32),
            jax.ShapeDtypeStruct((n, _D), jnp.float32),
            jax.ShapeDtypeStruct((n, _D), jnp.float32),
        ],
    )(z_pad, wx1_p, bx1, wx2, bx2, wx3, bx3, wpi, bpi, wdisp, bdisp, wmean, bmean)


def kernel(X, adj_n, W1, b1, W2, b2, mu, Wd, bd, Wb, Wx1, bx1, Wx2, bx2,
           Wx3, bx3, Wpi, bpi, Wdisp, bdisp, Wmean, bmean):
    f32 = jnp.float32

    w1c = W1.reshape(_K + 1, _D, _H).transpose(1, 0, 2).reshape(_D, (_K + 1) * _H)
    b1c = jnp.concatenate([b1, jnp.zeros((_K * _H,), f32)])
    w2c = jnp.zeros((_H, 4 * _LP), f32)
    for k in range(_K + 1):
        w2c = w2c.at[:, k * _LP:k * _LP + _L].set(W2[k * _H:(k + 1) * _H])
    b2c = jnp.zeros((4 * _LP,), f32).at[:_L].set(b2)
    mut = jnp.zeros((_LP, _LP), f32).at[:_L, :_NC].set(mu.T)
    wd_p = jnp.zeros((_LP, _ADJ), f32).at[:_L].set(Wd)
    wx1_p = jnp.zeros((_LP, 128), f32).at[:_L].set(Wx1)

    z_pad = _encoder(adj_n, X, w1c, b1c, w2c, b2c)

    q_out, m, hd = _zq(z_pad, mut, wd_p, bd, Wb)
    A_out = _aout(m, hd)

    pi, disp, mean = _decx(z_pad, wx1_p, bx1, Wx2, bx2, Wx3, bx3,
                           Wpi, bpi, Wdisp, bdisp, Wmean, bmean)

    return z_pad[:, :_L], q_out[:, :_NC], A_out, pi, disp, mean
